# baseline (speedup 1.0000x reference)
import functools

import numpy as np
import jax
from jax import lax
import jax.numpy as jnp
from jax.experimental import pallas as pl
from jax.experimental.pallas import tpu as pltpu
from jax.experimental.pallas import tpu_sc as plsc

B, H, S, DH = 1, 16, 2048, 64
W = 96
HALF = W // 2
G, TELE = 4, 2
G2 = G + TELE
TOP_U, PROTO = 12, 32
KQ_TOP = int(round(S * 0.2))
W_MEAN, W_MAX, W_TOPK, W_STD = 1.0, 0.6, 0.4, 0.2
SCALE = 1.0 / float(np.sqrt(DH))

QT = 256
KS = 384
TKC = 1024
NITER = 8
GP = 16
HPAIR = 16

PROTO_IDX = tuple(np.round(np.linspace(0, S - 1, PROTO)).astype(np.int32).tolist())


def _stats_kernel(q_ref, k_ref, u_ref, qp_ref):
    q = q_ref[0]
    k = k_ref[0]
    qn = q / jnp.maximum(jnp.sqrt(jnp.sum(q * q, axis=-1, keepdims=True)), 1e-6)
    kn = k / jnp.maximum(jnp.sqrt(jnp.sum(k * k, axis=-1, keepdims=True)), 1e-6)
    qp_ref[0] = jnp.concatenate([qn[i:i + 1, :] for i in PROTO_IDX], axis=0)

    for c in range(S // TKC):
        knc = kn[c * TKC:(c + 1) * TKC, :]
        sc = jax.lax.dot_general(qn, knc, (((1,), (1,)), ((), ())),
                                 preferred_element_type=jnp.float32)
        sc = jnp.maximum(sc, 0.0)
        total = jnp.sum(sc, axis=0, keepdims=True)
        mx = jnp.max(sc, axis=0, keepdims=True)
        sumsq = jnp.sum(sc * sc, axis=0, keepdims=True)

        def bis(_, carry):
            lo, hi = carry
            mid = 0.5 * (lo + hi)
            cnt = jnp.sum((sc > mid).astype(jnp.float32), axis=0, keepdims=True)
            pred = cnt >= KQ_TOP
            return jnp.where(pred, mid, lo), jnp.where(pred, hi, mid)

        lo0 = jnp.full_like(mx, -1e-3)
        hi0 = mx + 1e-6
        lo, hi = jax.lax.fori_loop(0, NITER, bis, (lo0, hi0))
        t_mid = 0.5 * (lo + hi)
        s_top = jnp.sum(jnp.maximum(sc - t_mid, 0.0), axis=0, keepdims=True)
        topk_mean = s_top / KQ_TOP + t_mid

        mean = total / S
        var = jnp.maximum(sumsq - total * total / S, 0.0) / (S - 1)
        u = W_MEAN * mean + W_MAX * mx + W_TOPK * topk_mean + W_STD * jnp.sqrt(var)
        u_ref[0, :, c * TKC:(c + 1) * TKC] = u


_GDN = lax.GatherDimensionNumbers(offset_dims=(), collapsed_slice_dims=(0,),
                                  start_index_map=(0,))


def _shuf(x, idx):
    return lax.gather(x, idx[:, None], _GDN, (1,),
                      mode=lax.GatherScatterMode.PROMISE_IN_BOUNDS)


def _allmax(x, lane):
    for sh in (8, 4, 2, 1):
        x = jnp.maximum(x, _shuf(x, (lane + sh) % 16))
    return x


def _allmin(x, lane):
    for sh in (8, 4, 2, 1):
        x = jnp.minimum(x, _shuf(x, (lane + sh) % 16))
    return x


def _allsum(x, lane):
    for sh in (8, 4, 2, 1):
        x = x + _shuf(x, (lane + sh) % 16)
    return x


def _route_body(u_hbm, qp_hbm, k_hbm, v_hbm, kg_hbm, vg_hbm,
                u_v, qp_v, ksub_v, gath_v, sem):
    nc = 2
    wid = lax.axis_index("s") * nc + lax.axis_index("c")

    @pl.when(wid < H)
    def _():
        h = wid
        base = h * S
        pltpu.sync_copy(u_hbm.at[h], u_v)
        pltpu.sync_copy(qp_hbm.at[pl.ds(h * PROTO, PROTO)], qp_v)
        lane = lax.iota(jnp.int32, 16)
        zero16 = jnp.zeros((16,), jnp.float32)
        for t in range(G2, GP):
            for c4 in range(DH // 16):
                gath_v[t, pl.ds(c4 * 16, 16)] = zero16
        top = []
        for _ in range(TOP_U):
            def mbody(i, m):
                return jnp.maximum(m, u_v[pl.ds(i * 16, 16)])

            mv = lax.fori_loop(0, S // 16, mbody,
                               jnp.full((16,), -3e38, jnp.float32))
            mv = _allmax(mv, lane)

            def abody(i, best):
                ch = u_v[pl.ds(i * 16, 16)]
                pos = jnp.where(ch >= mv, lane + i * 16, S)
                return jnp.minimum(best, pos)

            pv = lax.fori_loop(0, S // 16, abody,
                               jnp.full((16,), S, jnp.int32))
            j = _allmin(pv, lane)[0]
            top.append(j)
            cb = (j // 16) * 16
            ch = u_v[pl.ds(cb, 16)]
            u_v[pl.ds(cb, 16)] = jnp.where(lane == (j - cb), jnp.float32(-3e38), ch)

        cps = [pltpu.async_copy(k_hbm.at[h, top[t]],
                                ksub_v.at[t], sem) for t in range(TOP_U)]
        for c in cps:
            c.wait()

        s_rows = []
        for t in range(TOP_U):
            kr = [ksub_v[t, pl.ds(c4 * 16, 16)] for c4 in range(DH // 16)]
            sq = kr[0] * kr[0]
            for c4 in range(1, DH // 16):
                sq = sq + kr[c4] * kr[c4]
            nsq = jnp.maximum(_allsum(sq, lane), 1e-12)
            yi = 0x5F3759DF - (lax.bitcast_convert_type(nsq, jnp.int32) >> 1)
            y = lax.bitcast_convert_type(yi, jnp.float32)
            for _ in range(4):
                y = y * (1.5 - 0.5 * nsq * y * y)
            kr = [r * y for r in kr]
            halves = []
            for half in range(2):
                accv = jnp.zeros((16,), jnp.float32)
                for pi in range(16):
                    p = half * 16 + pi
                    pr = kr[0] * qp_v[p, pl.ds(0, 16)]
                    for c4 in range(1, DH // 16):
                        pr = pr + kr[c4] * qp_v[p, pl.ds(c4 * 16, 16)]
                    d = _allsum(pr, lane)[0]
                    accv = jnp.where(lane == pi, jnp.maximum(d, 0.0), accv)
                halves.append(accv)
            s_rows.append(halves)

        m0 = jnp.zeros((16,), jnp.float32)
        m1 = jnp.zeros((16,), jnp.float32)
        blocked = [jnp.int32(0) for _ in range(TOP_U)]
        chosen = []
        for _ in range(G):
            gains = []
            for t in range(TOP_U):
                gv = (jnp.maximum(s_rows[t][0] - m0, 0.0)
                      + jnp.maximum(s_rows[t][1] - m1, 0.0))
                g = _allsum(gv, lane)[0]
                gains.append(jnp.where(blocked[t] == 1, jnp.float32(-1e9), g))
            best_g = gains[0]
            best_t = jnp.int32(0)
            for t in range(1, TOP_U):
                better = gains[t] > best_g
                best_g = jnp.where(better, gains[t], best_g)
                best_t = jnp.where(better, jnp.int32(t), best_t)
            abs_i = jnp.int32(0)
            for t in range(TOP_U):
                is_t = best_t == t
                abs_i = jnp.where(is_t, top[t], abs_i)
                m0 = jnp.where(is_t, jnp.maximum(m0, s_rows[t][0]), m0)
                m1 = jnp.where(is_t, jnp.maximum(m1, s_rows[t][1]), m1)
                blocked[t] = jnp.where(is_t, jnp.int32(1), blocked[t])
            chosen.append(abs_i)

        routed = chosen + [h * 37, S // 2 + h * 37]

        cps = [pltpu.async_copy(k_hbm.at[h, routed[t]],
                                gath_v.at[t], sem) for t in range(G2)]
        for c in cps:
            c.wait()
        pltpu.sync_copy(gath_v, kg_hbm.at[pl.ds(h * GP, GP)])
        cps = [pltpu.async_copy(v_hbm.at[h, routed[t]],
                                gath_v.at[t], sem) for t in range(G2)]
        for c in cps:
            c.wait()
        pltpu.sync_copy(gath_v, vg_hbm.at[pl.ds(h * GP, GP)])


def _attn_kernel(q_ref, k_ref, v_ref, kg_ref, vg_ref, o_ref):
    jb = pl.program_id(1)
    q0 = jb * QT
    b = jnp.clip(q0 - HALF, 0, S - KS)
    t_abs = q0 + jax.lax.broadcasted_iota(jnp.int32, (QT, 1), 0)
    start = jnp.clip(t_abs - HALF, 0, S - W)
    a_abs = b + jax.lax.broadcasted_iota(jnp.int32, (1, KS), 1)
    band = (a_abs >= start) & (a_abs < start + W)
    gmask = jax.lax.broadcasted_iota(jnp.int32, (1, GP), 1) < G2

    outs = []
    for e in range(HPAIR):
        q = q_ref[e]
        ks = k_ref[e, pl.ds(b, KS), :]
        vs = v_ref[e, pl.ds(b, KS), :]

        s = jax.lax.dot_general(q, ks, (((1,), (1,)), ((), ())),
                                preferred_element_type=jnp.float32) * SCALE
        s = jnp.where(band, s, -1e30)

        kg = kg_ref[e]
        vg = vg_ref[e]
        sg = jax.lax.dot_general(q, kg, (((1,), (1,)), ((), ())),
                                 preferred_element_type=jnp.float32) * SCALE
        sg = jnp.where(gmask, sg, -1e30)

        m = jnp.maximum(jnp.max(s, axis=1, keepdims=True),
                        jnp.max(sg, axis=1, keepdims=True))
        p = jnp.exp(s - m)
        pg = jnp.exp(sg - m)
        denom = (jnp.sum(p, axis=1, keepdims=True)
                 + jnp.sum(pg, axis=1, keepdims=True))
        outs.append(
            (jax.lax.dot_general(p, vs, (((1,), (0,)), ((), ())),
                                 preferred_element_type=jnp.float32)
             + jax.lax.dot_general(pg, vg, (((1,), (0,)), ((), ())),
                                   preferred_element_type=jnp.float32))
            / denom)
    o_ref[...] = jnp.concatenate(outs, axis=1)


def kernel(q, k, v):
    q2 = q[0]
    k2 = k[0]
    v2 = v[0]

    u3, qp = pl.pallas_call(
        _stats_kernel,
        grid=(H,),
        in_specs=[
            pl.BlockSpec((1, S, DH), lambda h: (h, 0, 0)),
            pl.BlockSpec((1, S, DH), lambda h: (h, 0, 0)),
        ],
        out_specs=[
            pl.BlockSpec((1, 1, S), lambda h: (h, 0, 0)),
            pl.BlockSpec((1, PROTO, DH), lambda h: (h, 0, 0)),
        ],
        out_shape=[
            jax.ShapeDtypeStruct((H, 1, S), jnp.float32),
            jax.ShapeDtypeStruct((H, PROTO, DH), jnp.float32),
        ],
        compiler_params=pltpu.CompilerParams(
            dimension_semantics=("parallel",)),
    )(q2, k2)

    route = pl.kernel(
        _route_body,
        out_type=(
            jax.ShapeDtypeStruct((H * GP, DH), jnp.float32),
            jax.ShapeDtypeStruct((H * GP, DH), jnp.float32),
        ),
        mesh=plsc.VectorSubcoreMesh(core_axis_name="c", subcore_axis_name="s"),
        scratch_types=[
            pltpu.VMEM((S,), jnp.float32),
            pltpu.VMEM((PROTO, DH), jnp.float32),
            pltpu.VMEM((16, DH), jnp.float32),
            pltpu.VMEM((GP, DH), jnp.float32),
            pltpu.SemaphoreType.DMA,
        ],
    )
    kg, vg = route(u3.reshape(H, S), qp.reshape(H * PROTO, DH), k2, v2)
    kg3 = kg.reshape(H, GP, DH)
    vg3 = vg.reshape(H, GP, DH)

    out = pl.pallas_call(
        _attn_kernel,
        grid=(H // HPAIR, S // QT),
        in_specs=[
            pl.BlockSpec((HPAIR, QT, DH), lambda h, j: (h, j, 0)),
            pl.BlockSpec((HPAIR, S, DH), lambda h, j: (h, 0, 0)),
            pl.BlockSpec((HPAIR, S, DH), lambda h, j: (h, 0, 0)),
            pl.BlockSpec((HPAIR, GP, DH), lambda h, j: (h, 0, 0)),
            pl.BlockSpec((HPAIR, GP, DH), lambda h, j: (h, 0, 0)),
        ],
        out_specs=pl.BlockSpec((QT, HPAIR * DH), lambda h, j: (j, h)),
        out_shape=jax.ShapeDtypeStruct((S, H * DH), jnp.float32),
        compiler_params=pltpu.CompilerParams(
            dimension_semantics=("parallel", "arbitrary")),
    )(q2, k2, v2, kg3, vg3)

    return out.reshape(B, S, H * DH)


# speedup vs baseline: 1.0846x; 1.0846x over previous
import functools

import numpy as np
import jax
from jax import lax
import jax.numpy as jnp
from jax.experimental import pallas as pl
from jax.experimental.pallas import tpu as pltpu
from jax.experimental.pallas import tpu_sc as plsc

B, H, S, DH = 1, 16, 2048, 64
W = 96
HALF = W // 2
G, TELE = 4, 2
G2 = G + TELE
TOP_U, PROTO = 12, 32
KQ_TOP = int(round(S * 0.2))
W_MEAN, W_MAX, W_TOPK, W_STD = 1.0, 0.6, 0.4, 0.2
SCALE = 1.0 / float(np.sqrt(DH))

QT = 256
KS = 384
TKC = 512
NITER = 8
GP = 16
HPAIR = 16

PROTO_IDX = tuple(np.round(np.linspace(0, S - 1, PROTO)).astype(np.int32).tolist())


def _stats_kernel(q_ref, k_ref, u_ref, qp_ref):
    q = q_ref[0]
    k = k_ref[0]
    qn = q / jnp.maximum(jnp.sqrt(jnp.sum(q * q, axis=-1, keepdims=True)), 1e-6)
    kn = k / jnp.maximum(jnp.sqrt(jnp.sum(k * k, axis=-1, keepdims=True)), 1e-6)
    qp_ref[0] = jnp.concatenate([qn[i:i + 1, :] for i in PROTO_IDX], axis=0)

    for c in range(S // TKC):
        knc = kn[c * TKC:(c + 1) * TKC, :]
        sc = jax.lax.dot_general(qn, knc, (((1,), (1,)), ((), ())),
                                 preferred_element_type=jnp.float32)
        sc = jnp.maximum(sc, 0.0)
        total = jnp.sum(sc, axis=0, keepdims=True)
        mx = jnp.max(sc, axis=0, keepdims=True)
        sumsq = jnp.sum(sc * sc, axis=0, keepdims=True)

        def bis(_, carry):
            lo, hi = carry
            mid = 0.5 * (lo + hi)
            cnt = jnp.sum((sc > mid).astype(jnp.float32), axis=0, keepdims=True)
            pred = cnt >= KQ_TOP
            return jnp.where(pred, mid, lo), jnp.where(pred, hi, mid)

        lo0 = jnp.full_like(mx, -1e-3)
        hi0 = mx + 1e-6
        lo, hi = jax.lax.fori_loop(0, NITER, bis, (lo0, hi0))
        t_mid = 0.5 * (lo + hi)
        s_top = jnp.sum(jnp.maximum(sc - t_mid, 0.0), axis=0, keepdims=True)
        topk_mean = s_top / KQ_TOP + t_mid

        mean = total / S
        var = jnp.maximum(sumsq - total * total / S, 0.0) / (S - 1)
        u = W_MEAN * mean + W_MAX * mx + W_TOPK * topk_mean + W_STD * jnp.sqrt(var)
        u_ref[0, :, c * TKC:(c + 1) * TKC] = u


_GDN = lax.GatherDimensionNumbers(offset_dims=(), collapsed_slice_dims=(0,),
                                  start_index_map=(0,))


def _shuf(x, idx):
    return lax.gather(x, idx[:, None], _GDN, (1,),
                      mode=lax.GatherScatterMode.PROMISE_IN_BOUNDS)


def _allmax(x, lane):
    for sh in (8, 4, 2, 1):
        x = jnp.maximum(x, _shuf(x, (lane + sh) % 16))
    return x


def _allmin(x, lane):
    for sh in (8, 4, 2, 1):
        x = jnp.minimum(x, _shuf(x, (lane + sh) % 16))
    return x


def _allsum(x, lane):
    for sh in (8, 4, 2, 1):
        x = x + _shuf(x, (lane + sh) % 16)
    return x


def _route_body(u_hbm, qp_hbm, k_hbm, v_hbm, kg_hbm, vg_hbm,
                u_v, qp_v, ksub_v, gath_v, sem):
    nc = 2
    wid = lax.axis_index("s") * nc + lax.axis_index("c")

    @pl.when(wid < H)
    def _():
        h = wid
        base = h * S
        pltpu.sync_copy(u_hbm.at[h], u_v)
        pltpu.sync_copy(qp_hbm.at[pl.ds(h * PROTO, PROTO)], qp_v)
        lane = lax.iota(jnp.int32, 16)
        zero16 = jnp.zeros((16,), jnp.float32)
        for t in range(G2, GP):
            for c4 in range(DH // 16):
                gath_v[t, pl.ds(c4 * 16, 16)] = zero16
        top = []
        for _ in range(TOP_U):
            def mbody(i, m):
                return jnp.maximum(m, u_v[pl.ds(i * 16, 16)])

            mv = lax.fori_loop(0, S // 16, mbody,
                               jnp.full((16,), -3e38, jnp.float32))
            mv = _allmax(mv, lane)

            def abody(i, best):
                ch = u_v[pl.ds(i * 16, 16)]
                pos = jnp.where(ch >= mv, lane + i * 16, S)
                return jnp.minimum(best, pos)

            pv = lax.fori_loop(0, S // 16, abody,
                               jnp.full((16,), S, jnp.int32))
            j = _allmin(pv, lane)[0]
            top.append(j)
            cb = (j // 16) * 16
            ch = u_v[pl.ds(cb, 16)]
            u_v[pl.ds(cb, 16)] = jnp.where(lane == (j - cb), jnp.float32(-3e38), ch)

        cps = [pltpu.async_copy(k_hbm.at[h, top[t]],
                                ksub_v.at[t], sem) for t in range(TOP_U)]
        for c in cps:
            c.wait()

        s_rows = []
        for t in range(TOP_U):
            kr = [ksub_v[t, pl.ds(c4 * 16, 16)] for c4 in range(DH // 16)]
            sq = kr[0] * kr[0]
            for c4 in range(1, DH // 16):
                sq = sq + kr[c4] * kr[c4]
            nsq = jnp.maximum(_allsum(sq, lane), 1e-12)
            yi = 0x5F3759DF - (lax.bitcast_convert_type(nsq, jnp.int32) >> 1)
            y = lax.bitcast_convert_type(yi, jnp.float32)
            for _ in range(4):
                y = y * (1.5 - 0.5 * nsq * y * y)
            kr = [r * y for r in kr]
            halves = []
            for half in range(2):
                accv = jnp.zeros((16,), jnp.float32)
                for pi in range(16):
                    p = half * 16 + pi
                    pr = kr[0] * qp_v[p, pl.ds(0, 16)]
                    for c4 in range(1, DH // 16):
                        pr = pr + kr[c4] * qp_v[p, pl.ds(c4 * 16, 16)]
                    d = _allsum(pr, lane)[0]
                    accv = jnp.where(lane == pi, jnp.maximum(d, 0.0), accv)
                halves.append(accv)
            s_rows.append(halves)

        m0 = jnp.zeros((16,), jnp.float32)
        m1 = jnp.zeros((16,), jnp.float32)
        blocked = [jnp.int32(0) for _ in range(TOP_U)]
        chosen = []
        for _ in range(G):
            gains = []
            for t in range(TOP_U):
                gv = (jnp.maximum(s_rows[t][0] - m0, 0.0)
                      + jnp.maximum(s_rows[t][1] - m1, 0.0))
                g = _allsum(gv, lane)[0]
                gains.append(jnp.where(blocked[t] == 1, jnp.float32(-1e9), g))
            best_g = gains[0]
            best_t = jnp.int32(0)
            for t in range(1, TOP_U):
                better = gains[t] > best_g
                best_g = jnp.where(better, gains[t], best_g)
                best_t = jnp.where(better, jnp.int32(t), best_t)
            abs_i = jnp.int32(0)
            for t in range(TOP_U):
                is_t = best_t == t
                abs_i = jnp.where(is_t, top[t], abs_i)
                m0 = jnp.where(is_t, jnp.maximum(m0, s_rows[t][0]), m0)
                m1 = jnp.where(is_t, jnp.maximum(m1, s_rows[t][1]), m1)
                blocked[t] = jnp.where(is_t, jnp.int32(1), blocked[t])
            chosen.append(abs_i)

        routed = chosen + [h * 37, S // 2 + h * 37]

        cps = [pltpu.async_copy(k_hbm.at[h, routed[t]],
                                gath_v.at[t], sem) for t in range(G2)]
        for c in cps:
            c.wait()
        pltpu.sync_copy(gath_v, kg_hbm.at[pl.ds(h * GP, GP)])
        cps = [pltpu.async_copy(v_hbm.at[h, routed[t]],
                                gath_v.at[t], sem) for t in range(G2)]
        for c in cps:
            c.wait()
        pltpu.sync_copy(gath_v, vg_hbm.at[pl.ds(h * GP, GP)])


def _attn_kernel(q_ref, k_ref, v_ref, kg_ref, vg_ref, o_ref):
    jb = pl.program_id(1)
    q0 = jb * QT
    b = jnp.clip(q0 - HALF, 0, S - KS)
    t_abs = q0 + jax.lax.broadcasted_iota(jnp.int32, (QT, 1), 0)
    start = jnp.clip(t_abs - HALF, 0, S - W)
    a_abs = b + jax.lax.broadcasted_iota(jnp.int32, (1, KS), 1)
    band = (a_abs >= start) & (a_abs < start + W)
    gmask = jax.lax.broadcasted_iota(jnp.int32, (1, GP), 1) < G2

    outs = []
    for e in range(HPAIR):
        q = q_ref[e]
        ks = k_ref[e, pl.ds(b, KS), :]
        vs = v_ref[e, pl.ds(b, KS), :]

        s = jax.lax.dot_general(q, ks, (((1,), (1,)), ((), ())),
                                preferred_element_type=jnp.float32) * SCALE
        s = jnp.where(band, s, -1e30)

        kg = kg_ref[e]
        vg = vg_ref[e]
        sg = jax.lax.dot_general(q, kg, (((1,), (1,)), ((), ())),
                                 preferred_element_type=jnp.float32) * SCALE
        sg = jnp.where(gmask, sg, -1e30)

        m = jnp.maximum(jnp.max(s, axis=1, keepdims=True),
                        jnp.max(sg, axis=1, keepdims=True))
        p = jnp.exp(s - m)
        pg = jnp.exp(sg - m)
        denom = (jnp.sum(p, axis=1, keepdims=True)
                 + jnp.sum(pg, axis=1, keepdims=True))
        outs.append(
            (jax.lax.dot_general(p, vs, (((1,), (0,)), ((), ())),
                                 preferred_element_type=jnp.float32)
             + jax.lax.dot_general(pg, vg, (((1,), (0,)), ((), ())),
                                   preferred_element_type=jnp.float32))
            / denom)
    o_ref[...] = jnp.concatenate(outs, axis=1)


def kernel(q, k, v):
    q2 = q[0]
    k2 = k[0]
    v2 = v[0]

    u3, qp = pl.pallas_call(
        _stats_kernel,
        grid=(H,),
        in_specs=[
            pl.BlockSpec((1, S, DH), lambda h: (h, 0, 0)),
            pl.BlockSpec((1, S, DH), lambda h: (h, 0, 0)),
        ],
        out_specs=[
            pl.BlockSpec((1, 1, S), lambda h: (h, 0, 0)),
            pl.BlockSpec((1, PROTO, DH), lambda h: (h, 0, 0)),
        ],
        out_shape=[
            jax.ShapeDtypeStruct((H, 1, S), jnp.float32),
            jax.ShapeDtypeStruct((H, PROTO, DH), jnp.float32),
        ],
        compiler_params=pltpu.CompilerParams(
            dimension_semantics=("parallel",)),
    )(q2, k2)

    route = pl.kernel(
        _route_body,
        out_type=(
            jax.ShapeDtypeStruct((H * GP, DH), jnp.float32),
            jax.ShapeDtypeStruct((H * GP, DH), jnp.float32),
        ),
        mesh=plsc.VectorSubcoreMesh(core_axis_name="c", subcore_axis_name="s"),
        scratch_types=[
            pltpu.VMEM((S,), jnp.float32),
            pltpu.VMEM((PROTO, DH), jnp.float32),
            pltpu.VMEM((16, DH), jnp.float32),
            pltpu.VMEM((GP, DH), jnp.float32),
            pltpu.SemaphoreType.DMA,
        ],
    )
    kg, vg = route(u3.reshape(H, S), qp.reshape(H * PROTO, DH), k2, v2)
    kg3 = kg.reshape(H, GP, DH)
    vg3 = vg.reshape(H, GP, DH)

    out = pl.pallas_call(
        _attn_kernel,
        grid=(H // HPAIR, S // QT),
        in_specs=[
            pl.BlockSpec((HPAIR, QT, DH), lambda h, j: (h, j, 0)),
            pl.BlockSpec((HPAIR, S, DH), lambda h, j: (h, 0, 0)),
            pl.BlockSpec((HPAIR, S, DH), lambda h, j: (h, 0, 0)),
            pl.BlockSpec((HPAIR, GP, DH), lambda h, j: (h, 0, 0)),
            pl.BlockSpec((HPAIR, GP, DH), lambda h, j: (h, 0, 0)),
        ],
        out_specs=pl.BlockSpec((QT, HPAIR * DH), lambda h, j: (j, h)),
        out_shape=jax.ShapeDtypeStruct((S, H * DH), jnp.float32),
        compiler_params=pltpu.CompilerParams(
            dimension_semantics=("parallel", "arbitrary")),
    )(q2, k2, v2, kg3, vg3)

    return out.reshape(B, S, H * DH)
